# baseline (speedup 1.0000x reference)
import functools

import jax
import jax.numpy as jnp
from jax import lax
from jax.experimental import pallas as pl
from jax.experimental.pallas import tpu as pltpu
from jax.experimental.pallas import tpu_sc as plsc

_mesh = plsc.VectorSubcoreMesh(core_axis_name="c", subcore_axis_name="s")


@functools.partial(
    pl.kernel,
    out_type=jax.ShapeDtypeStruct((16,), jnp.float32),
    mesh=_mesh,
    scratch_types=[pltpu.VMEM((16,), jnp.float32)],
)
def _gather0(x_hbm, out_hbm, buf):
    wid = lax.axis_index("s") * 2 + lax.axis_index("c")

    @pl.when(wid == 0)
    def _():
        pltpu.sync_copy(x_hbm.at[pl.ds(0, 16)], buf)
        pltpu.sync_copy(buf, out_hbm)


def kernel(x):
    out = _gather0(x)
    v = out[0]
    return (v, v)


# speedup vs baseline: 1.0400x; 1.0400x over previous
import functools

import jax
import jax.numpy as jnp
from jax import lax
from jax.experimental import pallas as pl
from jax.experimental.pallas import tpu as pltpu
from jax.experimental.pallas import tpu_sc as plsc

_mesh = plsc.VectorSubcoreMesh(
    core_axis_name="c", subcore_axis_name="s", num_cores=1, num_subcores=1
)


@functools.partial(
    pl.kernel,
    out_type=jax.ShapeDtypeStruct((16,), jnp.float32),
    mesh=_mesh,
    scratch_types=[pltpu.VMEM((16,), jnp.float32)],
)
def _gather0(x_hbm, out_hbm, buf):
    pltpu.sync_copy(x_hbm.at[pl.ds(0, 16)], buf)
    pltpu.sync_copy(buf, out_hbm)


def kernel(x):
    out = _gather0(x)
    v = out[0]
    return (v, v)


# speedup vs baseline: 5.9547x; 5.7259x over previous
import jax
import jax.numpy as jnp
from jax.experimental import pallas as pl
from jax.experimental.pallas import tpu as pltpu


def _body(x_ref, o_ref):
    o_ref[0] = x_ref[0]


def kernel(x):
    o = pl.pallas_call(
        _body,
        grid=(1,),
        in_specs=[pl.BlockSpec((128,), lambda i: (0,))],
        out_specs=pl.BlockSpec(memory_space=pltpu.SMEM),
        out_shape=jax.ShapeDtypeStruct((1,), jnp.float32),
    )(x)
    v = o[0]
    return (v, v)


# speedup vs baseline: 10.9787x; 1.8437x over previous
import jax
import jax.numpy as jnp
from jax.experimental import pallas as pl
from jax.experimental.pallas import tpu as pltpu


def _body(x_ref, a_ref, b_ref):
    v = x_ref[0]
    a_ref[0] = v
    b_ref[0] = v


def kernel(x):
    a, b = pl.pallas_call(
        _body,
        grid=(1,),
        in_specs=[pl.BlockSpec((128,), lambda i: (0,))],
        out_specs=(pl.BlockSpec(memory_space=pltpu.SMEM),
                   pl.BlockSpec(memory_space=pltpu.SMEM)),
        out_shape=(jax.ShapeDtypeStruct((1,), jnp.float32),
                   jax.ShapeDtypeStruct((1,), jnp.float32)),
    )(x)
    return (a.reshape(()), b.reshape(()))
